# baseline (speedup 1.0000x reference)
import functools

import jax
import jax.numpy as jnp
from jax import lax
from jax.experimental import pallas as pl
from jax.experimental.pallas import tpu as pltpu
from jax.experimental.pallas import tpu_sc as plsc

_BM = 256


def _route_meta(token_per_expert, n_rows, n_experts):
    nb = n_rows // _BM
    t_max = nb + n_experts - 1
    off = jnp.concatenate([
        jnp.zeros((1,), jnp.int32),
        jnp.cumsum(token_per_expert).astype(jnp.int32),
    ])
    b = jnp.arange(nb, dtype=jnp.int32)[:, None]
    seg_lo = jnp.maximum(off[:-1][None, :], b * _BM)
    seg_hi = jnp.minimum(off[1:][None, :], (b + 1) * _BM)
    valid = seg_lo < seg_hi
    vflat = valid.reshape(-1)
    slot = jnp.where(vflat, jnp.cumsum(vflat) - 1, t_max).astype(jnp.int32)

    def scat(vals, fill):
        buf = jnp.full((t_max + 1,), fill, jnp.int32)
        return buf.at[slot].set(vals.reshape(-1).astype(jnp.int32))[:t_max]

    blk = scat(jnp.broadcast_to(b, valid.shape), nb - 1)
    expt = scat(jnp.broadcast_to(jnp.arange(n_experts, dtype=jnp.int32)[None, :],
                                 valid.shape), n_experts - 1)
    lo = scat(seg_lo - b * _BM, 0)
    hi = scat(seg_hi - b * _BM, 0)
    first = scat(valid & (jnp.cumsum(valid, axis=1) == 1), 0)
    return blk, expt, lo, hi, first


def _tc_grouped_mlp(xg, probs_sorted, w_gate, w_up, w_down,
                    blk, expt, lo, hi, first):
    n_rows, h = xg.shape
    n_experts, ff, _ = w_gate.shape
    nb = n_rows // _BM
    t_max = nb + n_experts - 1
    probs3 = probs_sorted.reshape(nb, 1, _BM)

    def body(blk_r, expt_r, lo_r, hi_r, first_r, x_r, p_r, wg_r, wu_r, wd_r,
             o_r):
        t = pl.program_id(0)
        rows = lax.broadcasted_iota(jnp.int32, (_BM, 1), 0)
        mask = (rows >= lo_r[t]) & (rows < hi_r[t])
        x = x_r[...]
        g = lax.dot_general(x, wg_r[0], (((1,), (1,)), ((), ())),
                            preferred_element_type=jnp.float32)
        u = lax.dot_general(x, wu_r[0], (((1,), (1,)), ((), ())),
                            preferred_element_type=jnp.float32)
        act = u * (g * jax.nn.sigmoid(g))
        d = lax.dot_general(act, wd_r[0], (((1,), (1,)), ((), ())),
                            preferred_element_type=jnp.float32)
        d = jnp.where(mask, d * p_r[0, 0, :].reshape(_BM, 1), 0.0)
        is_init = first_r[t] == 1

        @pl.when(is_init)
        def _():
            o_r[...] = d

        @pl.when(jnp.logical_not(is_init))
        def _():
            o_r[...] += d

    grid_spec = pltpu.PrefetchScalarGridSpec(
        num_scalar_prefetch=5,
        grid=(t_max,),
        in_specs=[
            pl.BlockSpec((_BM, h), lambda t, blk, expt, lo, hi, first: (blk[t], 0)),
            pl.BlockSpec((1, 1, _BM), lambda t, blk, expt, lo, hi, first: (blk[t], 0, 0)),
            pl.BlockSpec((1, ff, h), lambda t, blk, expt, lo, hi, first: (expt[t], 0, 0)),
            pl.BlockSpec((1, ff, h), lambda t, blk, expt, lo, hi, first: (expt[t], 0, 0)),
            pl.BlockSpec((1, h, ff), lambda t, blk, expt, lo, hi, first: (expt[t], 0, 0)),
        ],
        out_specs=pl.BlockSpec((_BM, h), lambda t, blk, expt, lo, hi, first: (blk[t], 0)),
    )
    return pl.pallas_call(
        body,
        grid_spec=grid_spec,
        out_shape=jax.ShapeDtypeStruct((n_rows, h), jnp.float32),
        compiler_params=pltpu.CompilerParams(
            dimension_semantics=("arbitrary",)),
    )(blk, expt, lo, hi, first, xg, probs3, w_gate, w_up, w_down)


def _sc_gather_rows(table, idx):
    n_rows = idx.shape[0]
    h = table.shape[1]
    info = plsc.get_sparse_core_info()
    nw = info.num_cores * info.num_subcores
    bpw = n_rows // nw
    ch = min(32, bpw)
    nch = bpw // ch
    idx3 = idx.reshape(nw, nch, ch)
    mesh = plsc.VectorSubcoreMesh(core_axis_name="c", subcore_axis_name="s")

    @functools.partial(
        pl.kernel,
        mesh=mesh,
        out_type=jax.ShapeDtypeStruct((n_rows, h), jnp.float32),
        scratch_types=[
            pltpu.VMEM((nch, ch), jnp.int32),
            pltpu.VMEM((ch, h), jnp.float32),
            pltpu.VMEM((ch, h), jnp.float32),
            pltpu.SemaphoreType.DMA,
            pltpu.SemaphoreType.DMA,
        ],
    )
    def k(table_hbm, idx_hbm, out_hbm, idx_v, r0, r1, s0, s1):
        wid = lax.axis_index("s") * info.num_cores + lax.axis_index("c")
        base = wid * bpw
        pltpu.sync_copy(idx_hbm.at[wid], idx_v)
        bufs = (r0, r1)
        sems = (s0, s1)
        cps = [None] * nch
        cps[0] = pltpu.async_copy(table_hbm.at[idx_v.at[0]], bufs[0], sems[0])
        for c in range(nch):
            cps[c].wait()
            if c + 1 < nch:
                cps[c + 1] = pltpu.async_copy(
                    table_hbm.at[idx_v.at[c + 1]], bufs[(c + 1) % 2],
                    sems[(c + 1) % 2])
            pltpu.sync_copy(bufs[c % 2], out_hbm.at[pl.ds(base + c * ch, ch)])

    return k(table, idx3)


def _sc_combine(down, pos_a, pos_b):
    s = pos_a.shape[0]
    h = down.shape[1]
    info = plsc.get_sparse_core_info()
    nw = info.num_cores * info.num_subcores
    bpw = s // nw
    ch = min(16, bpw)
    nch = bpw // ch
    nvec = h // 16
    pa3 = pos_a.reshape(nw, nch, ch)
    pb3 = pos_b.reshape(nw, nch, ch)
    mesh = plsc.VectorSubcoreMesh(core_axis_name="c", subcore_axis_name="s")

    @functools.partial(
        pl.kernel,
        mesh=mesh,
        out_type=jax.ShapeDtypeStruct((s, h), jnp.float32),
        scratch_types=[
            pltpu.VMEM((nch, ch), jnp.int32),
            pltpu.VMEM((nch, ch), jnp.int32),
            pltpu.VMEM((ch, h), jnp.float32),
            pltpu.VMEM((ch, h), jnp.float32),
            pltpu.VMEM((ch, h), jnp.float32),
            pltpu.VMEM((ch, h), jnp.float32),
            pltpu.SemaphoreType.DMA,
            pltpu.SemaphoreType.DMA,
            pltpu.SemaphoreType.DMA,
            pltpu.SemaphoreType.DMA,
        ],
    )
    def k(down_hbm, pa_hbm, pb_hbm, out_hbm, ia, ib,
          ra0, rb0, ra1, rb1, sa0, sb0, sa1, sb1):
        wid = lax.axis_index("s") * info.num_cores + lax.axis_index("c")
        base = wid * bpw
        pltpu.sync_copy(pa_hbm.at[wid], ia)
        pltpu.sync_copy(pb_hbm.at[wid], ib)
        ras = (ra0, ra1)
        rbs = (rb0, rb1)
        sas = (sa0, sa1)
        sbs = (sb0, sb1)
        cpa = [None] * nch
        cpb = [None] * nch
        cpa[0] = pltpu.async_copy(down_hbm.at[ia.at[0]], ras[0], sas[0])
        cpb[0] = pltpu.async_copy(down_hbm.at[ib.at[0]], rbs[0], sbs[0])
        for c in range(nch):
            cpa[c].wait()
            cpb[c].wait()
            if c + 1 < nch:
                j = (c + 1) % 2
                cpa[c + 1] = pltpu.async_copy(down_hbm.at[ia.at[c + 1]], ras[j], sas[j])
                cpb[c + 1] = pltpu.async_copy(down_hbm.at[ib.at[c + 1]], rbs[j], sbs[j])
            ra = ras[c % 2]
            rb = rbs[c % 2]

            def add_row(r, carry):
                for j in range(nvec):
                    ra[r, pl.ds(j * 16, 16)] = (
                        ra[r, pl.ds(j * 16, 16)] + rb[r, pl.ds(j * 16, 16)])
                return carry

            lax.fori_loop(0, ch, add_row, 0)
            pltpu.sync_copy(ra, out_hbm.at[pl.ds(base + c * ch, ch)])

    return k(down, pa3, pb3)


def kernel(hidden_states, router_weights, selected_experts, token_per_expert,
           W_gate, W_up, W_down):
    s, h = hidden_states.shape
    k_ = router_weights.shape[1]
    n_experts = W_gate.shape[0]
    n_rows = k_ * s
    nb = n_rows // _BM

    dup = selected_experts.T.reshape(-1)
    arange_r = jnp.arange(n_rows, dtype=jnp.int32)
    onehot_t = (dup[None, :] == jnp.arange(n_experts, dtype=dup.dtype)[:, None]
                ).astype(jnp.int32)
    rank_incl = jnp.sum(jnp.cumsum(onehot_t, axis=1) * onehot_t, axis=0)
    off0 = jnp.concatenate([
        jnp.zeros((1,), jnp.int32),
        jnp.cumsum(token_per_expert).astype(jnp.int32)[:-1],
    ])
    pos = jnp.take(off0, dup) + rank_incl - 1
    src_token = jnp.zeros((n_rows,), jnp.int32).at[pos].set(arange_r % s)
    probs_sorted = jnp.zeros((n_rows,), jnp.float32).at[pos].set(
        router_weights.T.reshape(-1).astype(jnp.float32))
    blk, expt, lo, hi, first = _route_meta(token_per_expert, n_rows, n_experts)

    grouped = _sc_gather_rows(hidden_states, src_token)
    down = _tc_grouped_mlp(grouped, probs_sorted, W_gate, W_up, W_down,
                           blk, expt, lo, hi, first)
    return _sc_combine(down, pos[:s], pos[s:])


# speedup vs baseline: 1.1702x; 1.1702x over previous
import functools

import jax
import jax.numpy as jnp
from jax import lax
from jax.experimental import pallas as pl
from jax.experimental.pallas import tpu as pltpu
from jax.experimental.pallas import tpu_sc as plsc

_BM = 256


def _route_meta(token_per_expert, n_rows, n_experts):
    nb = n_rows // _BM
    t_max = nb + n_experts - 1
    off = jnp.concatenate([
        jnp.zeros((1,), jnp.int32),
        jnp.cumsum(token_per_expert).astype(jnp.int32),
    ])
    b = jnp.arange(nb, dtype=jnp.int32)[:, None]
    seg_lo = jnp.maximum(off[:-1][None, :], b * _BM)
    seg_hi = jnp.minimum(off[1:][None, :], (b + 1) * _BM)
    valid = seg_lo < seg_hi
    vflat = valid.reshape(-1)
    slot = jnp.where(vflat, jnp.cumsum(vflat) - 1, t_max).astype(jnp.int32)

    def scat(vals, fill):
        buf = jnp.full((t_max + 1,), fill, jnp.int32)
        return buf.at[slot].set(vals.reshape(-1).astype(jnp.int32))[:t_max]

    blk = scat(jnp.broadcast_to(b, valid.shape), nb - 1)
    expt = scat(jnp.broadcast_to(jnp.arange(n_experts, dtype=jnp.int32)[None, :],
                                 valid.shape), n_experts - 1)
    lo = scat(seg_lo - b * _BM, 0)
    hi = scat(seg_hi - b * _BM, 0)
    first = scat(valid & (jnp.cumsum(valid, axis=1) == 1), 0)
    return blk, expt, lo, hi, first


def _tc_grouped_mlp(xg, probs_sorted, w_gate, w_up, w_down,
                    blk, expt, lo, hi, first):
    n_rows, h = xg.shape
    n_experts, ff, _ = w_gate.shape
    nb = n_rows // _BM
    t_max = nb + n_experts - 1
    probs3 = probs_sorted.reshape(nb, 1, _BM)

    def body(blk_r, expt_r, lo_r, hi_r, first_r, x_r, p_r, wg_r, wu_r, wd_r,
             o_r):
        t = pl.program_id(0)
        rows = lax.broadcasted_iota(jnp.int32, (_BM, 1), 0)
        mask = (rows >= lo_r[t]) & (rows < hi_r[t])
        x = x_r[...]
        g = lax.dot_general(x, wg_r[0], (((1,), (1,)), ((), ())),
                            preferred_element_type=jnp.float32)
        u = lax.dot_general(x, wu_r[0], (((1,), (1,)), ((), ())),
                            preferred_element_type=jnp.float32)
        act = u * (g * jax.nn.sigmoid(g))
        d = lax.dot_general(act, wd_r[0], (((1,), (1,)), ((), ())),
                            preferred_element_type=jnp.float32)
        d = jnp.where(mask, d * p_r[0, 0, :].reshape(_BM, 1), 0.0)
        is_init = first_r[t] == 1

        @pl.when(is_init)
        def _():
            o_r[...] = d

        @pl.when(jnp.logical_not(is_init))
        def _():
            o_r[...] += d

    grid_spec = pltpu.PrefetchScalarGridSpec(
        num_scalar_prefetch=5,
        grid=(t_max,),
        in_specs=[
            pl.BlockSpec((_BM, h), lambda t, blk, expt, lo, hi, first: (blk[t], 0)),
            pl.BlockSpec((1, 1, _BM), lambda t, blk, expt, lo, hi, first: (blk[t], 0, 0)),
            pl.BlockSpec((1, ff, h), lambda t, blk, expt, lo, hi, first: (expt[t], 0, 0)),
            pl.BlockSpec((1, ff, h), lambda t, blk, expt, lo, hi, first: (expt[t], 0, 0)),
            pl.BlockSpec((1, h, ff), lambda t, blk, expt, lo, hi, first: (expt[t], 0, 0)),
        ],
        out_specs=pl.BlockSpec((_BM, h), lambda t, blk, expt, lo, hi, first: (blk[t], 0)),
    )
    return pl.pallas_call(
        body,
        grid_spec=grid_spec,
        out_shape=jax.ShapeDtypeStruct((n_rows, h), jnp.float32),
        compiler_params=pltpu.CompilerParams(
            dimension_semantics=("arbitrary",)),
    )(blk, expt, lo, hi, first, xg, probs3, w_gate, w_up, w_down)


def _sc_gather_rows(table, idx):
    n_rows = idx.shape[0]
    h = table.shape[1]
    info = plsc.get_sparse_core_info()
    nw = info.num_cores * info.num_subcores
    bpw = n_rows // nw
    ch = min(32, bpw)
    nch = bpw // ch
    idx3 = idx.reshape(nw, nch, ch)
    mesh = plsc.VectorSubcoreMesh(core_axis_name="c", subcore_axis_name="s")

    @functools.partial(
        pl.kernel,
        mesh=mesh,
        out_type=jax.ShapeDtypeStruct((n_rows, h), jnp.float32),
        scratch_types=[
            pltpu.VMEM((nch, ch), jnp.int32),
            pltpu.VMEM((ch, h), jnp.float32),
            pltpu.VMEM((ch, h), jnp.float32),
            pltpu.SemaphoreType.DMA,
            pltpu.SemaphoreType.DMA,
        ],
    )
    def k(table_hbm, idx_hbm, out_hbm, idx_v, r0, r1, s0, s1):
        wid = lax.axis_index("s") * info.num_cores + lax.axis_index("c")
        base = wid * bpw
        pltpu.sync_copy(idx_hbm.at[wid], idx_v)
        bufs = (r0, r1)
        sems = (s0, s1)
        cps = [None] * nch
        cps[0] = pltpu.async_copy(table_hbm.at[idx_v.at[0]], bufs[0], sems[0])
        for c in range(nch):
            cps[c].wait()
            if c + 1 < nch:
                cps[c + 1] = pltpu.async_copy(
                    table_hbm.at[idx_v.at[c + 1]], bufs[(c + 1) % 2],
                    sems[(c + 1) % 2])
            pltpu.sync_copy(bufs[c % 2], out_hbm.at[pl.ds(base + c * ch, ch)])

    return k(table, idx3)


def _sc_combine(down, pos_a, pos_b):
    s = pos_a.shape[0]
    h = down.shape[1]
    info = plsc.get_sparse_core_info()
    nw = info.num_cores * info.num_subcores
    bpw = s // nw
    ch = min(16, bpw)
    nch = bpw // ch
    nvec = h // 16
    pa3 = pos_a.reshape(nw, nch, ch)
    pb3 = pos_b.reshape(nw, nch, ch)
    mesh = plsc.VectorSubcoreMesh(core_axis_name="c", subcore_axis_name="s")

    @functools.partial(
        pl.kernel,
        mesh=mesh,
        out_type=jax.ShapeDtypeStruct((s, h), jnp.float32),
        scratch_types=[
            pltpu.VMEM((nch, ch), jnp.int32),
            pltpu.VMEM((nch, ch), jnp.int32),
            pltpu.VMEM((ch, h), jnp.float32),
            pltpu.VMEM((ch, h), jnp.float32),
            pltpu.VMEM((ch, h), jnp.float32),
            pltpu.VMEM((ch, h), jnp.float32),
            pltpu.SemaphoreType.DMA,
            pltpu.SemaphoreType.DMA,
            pltpu.SemaphoreType.DMA,
            pltpu.SemaphoreType.DMA,
        ],
    )
    def k(down_hbm, pa_hbm, pb_hbm, out_hbm, ia, ib,
          ra0, rb0, ra1, rb1, sa0, sb0, sa1, sb1):
        wid = lax.axis_index("s") * info.num_cores + lax.axis_index("c")
        base = wid * bpw
        pltpu.sync_copy(pa_hbm.at[wid], ia)
        pltpu.sync_copy(pb_hbm.at[wid], ib)
        ras = (ra0, ra1)
        rbs = (rb0, rb1)
        sas = (sa0, sa1)
        sbs = (sb0, sb1)
        cpa = [None] * nch
        cpb = [None] * nch
        cpa[0] = pltpu.async_copy(down_hbm.at[ia.at[0]], ras[0], sas[0])
        cpb[0] = pltpu.async_copy(down_hbm.at[ib.at[0]], rbs[0], sbs[0])
        for c in range(nch):
            cpa[c].wait()
            cpb[c].wait()
            if c + 1 < nch:
                j = (c + 1) % 2
                cpa[c + 1] = pltpu.async_copy(down_hbm.at[ia.at[c + 1]], ras[j], sas[j])
                cpb[c + 1] = pltpu.async_copy(down_hbm.at[ib.at[c + 1]], rbs[j], sbs[j])
            ra = ras[c % 2]
            rb = rbs[c % 2]

            def add_row(r, carry):
                for j in range(nvec):
                    ra[r, pl.ds(j * 16, 16)] = (
                        ra[r, pl.ds(j * 16, 16)] + rb[r, pl.ds(j * 16, 16)])
                return carry

            lax.fori_loop(0, ch, add_row, 0)
            pltpu.sync_copy(ra, out_hbm.at[pl.ds(base + c * ch, ch)])

    return k(down, pa3, pb3)


def kernel(hidden_states, router_weights, selected_experts, token_per_expert,
           W_gate, W_up, W_down):
    s, h = hidden_states.shape
    k_ = router_weights.shape[1]
    n_experts = W_gate.shape[0]
    n_rows = k_ * s
    nb = n_rows // _BM

    dup = selected_experts.T.reshape(-1)
    sort_idx = jnp.argsort(dup, stable=True).astype(jnp.int32)
    pos = jnp.argsort(sort_idx).astype(jnp.int32)
    src_token = (sort_idx % s).astype(jnp.int32)
    probs_sorted = jnp.take(router_weights.T.reshape(-1).astype(jnp.float32),
                            sort_idx)
    blk, expt, lo, hi, first = _route_meta(token_per_expert, n_rows, n_experts)

    grouped = _sc_gather_rows(hidden_states, src_token)
    down = _tc_grouped_mlp(grouped, probs_sorted, W_gate, W_up, W_down,
                           blk, expt, lo, hi, first)
    return _sc_combine(down, pos[:s], pos[s:])


# speedup vs baseline: 1.1865x; 1.0139x over previous
import functools

import jax
import jax.numpy as jnp
from jax import lax
from jax.experimental import pallas as pl
from jax.experimental.pallas import tpu as pltpu
from jax.experimental.pallas import tpu_sc as plsc

_BM = 256


def _route_meta(token_per_expert, n_rows, n_experts):
    nb = n_rows // _BM
    t_max = nb + n_experts - 1
    off = jnp.concatenate([
        jnp.zeros((1,), jnp.int32),
        jnp.cumsum(token_per_expert).astype(jnp.int32),
    ])
    b = jnp.arange(nb, dtype=jnp.int32)[:, None]
    seg_lo = jnp.maximum(off[:-1][None, :], b * _BM)
    seg_hi = jnp.minimum(off[1:][None, :], (b + 1) * _BM)
    valid = seg_lo < seg_hi
    vflat = valid.reshape(-1)
    slot = jnp.where(vflat, jnp.cumsum(vflat) - 1, t_max).astype(jnp.int32)

    def scat(vals, fill):
        buf = jnp.full((t_max + 1,), fill, jnp.int32)
        return buf.at[slot].set(vals.reshape(-1).astype(jnp.int32))[:t_max]

    blk = scat(jnp.broadcast_to(b, valid.shape), nb - 1)
    expt = scat(jnp.broadcast_to(jnp.arange(n_experts, dtype=jnp.int32)[None, :],
                                 valid.shape), n_experts - 1)
    lo = scat(seg_lo - b * _BM, 0)
    hi = scat(seg_hi - b * _BM, 0)
    first = scat(valid & (jnp.cumsum(valid, axis=1) == 1), 0)
    return blk, expt, lo, hi, first


def _tc_grouped_mlp(xg, probs_sorted, w_gate, w_up, w_down,
                    blk, expt, lo, hi, first):
    n_rows, h = xg.shape
    n_experts, ff, _ = w_gate.shape
    nb = n_rows // _BM
    t_max = nb + n_experts - 1
    probs3 = probs_sorted.reshape(nb, 1, _BM)

    def body(blk_r, expt_r, lo_r, hi_r, first_r, x_r, p_r, wg_r, wu_r, wd_r,
             o_r):
        t = pl.program_id(0)
        rows = lax.broadcasted_iota(jnp.int32, (_BM, 1), 0)
        mask = (rows >= lo_r[t]) & (rows < hi_r[t])
        x = x_r[...]
        g = lax.dot_general(x, wg_r[0], (((1,), (1,)), ((), ())),
                            preferred_element_type=jnp.float32)
        u = lax.dot_general(x, wu_r[0], (((1,), (1,)), ((), ())),
                            preferred_element_type=jnp.float32)
        act = u * (g * jax.nn.sigmoid(g))
        d = lax.dot_general(act, wd_r[0], (((1,), (1,)), ((), ())),
                            preferred_element_type=jnp.float32)
        d = jnp.where(mask, d * p_r[0, 0, :].reshape(_BM, 1), 0.0)
        is_init = first_r[t] == 1

        @pl.when(is_init)
        def _():
            o_r[...] = d

        @pl.when(jnp.logical_not(is_init))
        def _():
            o_r[...] += d

    grid_spec = pltpu.PrefetchScalarGridSpec(
        num_scalar_prefetch=5,
        grid=(t_max,),
        in_specs=[
            pl.BlockSpec((_BM, h), lambda t, blk, expt, lo, hi, first: (blk[t], 0)),
            pl.BlockSpec((1, 1, _BM), lambda t, blk, expt, lo, hi, first: (blk[t], 0, 0)),
            pl.BlockSpec((1, ff, h), lambda t, blk, expt, lo, hi, first: (expt[t], 0, 0)),
            pl.BlockSpec((1, ff, h), lambda t, blk, expt, lo, hi, first: (expt[t], 0, 0)),
            pl.BlockSpec((1, h, ff), lambda t, blk, expt, lo, hi, first: (expt[t], 0, 0)),
        ],
        out_specs=pl.BlockSpec((_BM, h), lambda t, blk, expt, lo, hi, first: (blk[t], 0)),
    )
    return pl.pallas_call(
        body,
        grid_spec=grid_spec,
        out_shape=jax.ShapeDtypeStruct((n_rows, h), jnp.float32),
        compiler_params=pltpu.CompilerParams(
            dimension_semantics=("arbitrary",)),
    )(blk, expt, lo, hi, first, xg, probs3, w_gate, w_up, w_down)


def _sc_gather_rows(table, idx):
    n_rows = idx.shape[0]
    h = table.shape[1]
    info = plsc.get_sparse_core_info()
    nw = info.num_cores * info.num_subcores
    bpw = n_rows // nw
    ch = min(64, bpw)
    nch = bpw // ch
    idx3 = idx.reshape(nw, nch, ch)
    mesh = plsc.VectorSubcoreMesh(core_axis_name="c", subcore_axis_name="s")

    @functools.partial(
        pl.kernel,
        mesh=mesh,
        out_type=jax.ShapeDtypeStruct((n_rows, h), jnp.float32),
        scratch_types=[
            pltpu.VMEM((nch, ch), jnp.int32),
            pltpu.VMEM((ch, h), jnp.float32),
            pltpu.SemaphoreType.DMA,
        ],
    )
    def k(table_hbm, idx_hbm, out_hbm, idx_v, rows_v, sem):
        wid = lax.axis_index("s") * info.num_cores + lax.axis_index("c")
        base = wid * bpw
        pltpu.sync_copy(idx_hbm.at[wid], idx_v)
        for c in range(nch):
            pltpu.async_copy(table_hbm.at[idx_v.at[c]], rows_v, sem).wait()
            pltpu.sync_copy(rows_v, out_hbm.at[pl.ds(base + c * ch, ch)])

    return k(table, idx3)


def _sc_combine(down, pos_a, pos_b):
    s = pos_a.shape[0]
    h = down.shape[1]
    info = plsc.get_sparse_core_info()
    nw = info.num_cores * info.num_subcores
    bpw = s // nw
    ch = min(16, bpw)
    nch = bpw // ch
    nvec = h // 16
    pa3 = pos_a.reshape(nw, nch, ch)
    pb3 = pos_b.reshape(nw, nch, ch)
    mesh = plsc.VectorSubcoreMesh(core_axis_name="c", subcore_axis_name="s")

    @functools.partial(
        pl.kernel,
        mesh=mesh,
        out_type=jax.ShapeDtypeStruct((s, h), jnp.float32),
        scratch_types=[
            pltpu.VMEM((nch, ch), jnp.int32),
            pltpu.VMEM((nch, ch), jnp.int32),
            pltpu.VMEM((ch, h), jnp.float32),
            pltpu.VMEM((ch, h), jnp.float32),
            pltpu.VMEM((ch, h), jnp.float32),
            pltpu.VMEM((ch, h), jnp.float32),
            pltpu.SemaphoreType.DMA,
            pltpu.SemaphoreType.DMA,
            pltpu.SemaphoreType.DMA,
            pltpu.SemaphoreType.DMA,
        ],
    )
    def k(down_hbm, pa_hbm, pb_hbm, out_hbm, ia, ib,
          ra0, rb0, ra1, rb1, sa0, sb0, sa1, sb1):
        wid = lax.axis_index("s") * info.num_cores + lax.axis_index("c")
        base = wid * bpw
        pltpu.sync_copy(pa_hbm.at[wid], ia)
        pltpu.sync_copy(pb_hbm.at[wid], ib)
        ras = (ra0, ra1)
        rbs = (rb0, rb1)
        sas = (sa0, sa1)
        sbs = (sb0, sb1)
        cpa = [None] * nch
        cpb = [None] * nch
        cpa[0] = pltpu.async_copy(down_hbm.at[ia.at[0]], ras[0], sas[0])
        cpb[0] = pltpu.async_copy(down_hbm.at[ib.at[0]], rbs[0], sbs[0])
        for c in range(nch):
            cpa[c].wait()
            cpb[c].wait()
            if c + 1 < nch:
                j = (c + 1) % 2
                cpa[c + 1] = pltpu.async_copy(down_hbm.at[ia.at[c + 1]], ras[j], sas[j])
                cpb[c + 1] = pltpu.async_copy(down_hbm.at[ib.at[c + 1]], rbs[j], sbs[j])
            ra = ras[c % 2]
            rb = rbs[c % 2]

            def add_row(r, carry):
                for j in range(nvec):
                    ra[r, pl.ds(j * 16, 16)] = (
                        ra[r, pl.ds(j * 16, 16)] + rb[r, pl.ds(j * 16, 16)])
                return carry

            lax.fori_loop(0, ch, add_row, 0)
            pltpu.sync_copy(ra, out_hbm.at[pl.ds(base + c * ch, ch)])

    return k(down, pa3, pb3)


def kernel(hidden_states, router_weights, selected_experts, token_per_expert,
           W_gate, W_up, W_down):
    s, h = hidden_states.shape
    k_ = router_weights.shape[1]
    n_experts = W_gate.shape[0]
    n_rows = k_ * s
    nb = n_rows // _BM

    dup = selected_experts.T.reshape(-1)
    sort_idx = jnp.argsort(dup, stable=True).astype(jnp.int32)
    pos = jnp.argsort(sort_idx).astype(jnp.int32)
    src_token = (sort_idx % s).astype(jnp.int32)
    probs_sorted = jnp.take(router_weights.T.reshape(-1).astype(jnp.float32),
                            sort_idx)
    blk, expt, lo, hi, first = _route_meta(token_per_expert, n_rows, n_experts)

    grouped = _sc_gather_rows(hidden_states, src_token)
    down = _tc_grouped_mlp(grouped, probs_sorted, W_gate, W_up, W_down,
                           blk, expt, lo, hi, first)
    return _sc_combine(down, pos[:s], pos[s:])


# speedup vs baseline: 1.2088x; 1.0189x over previous
import functools

import jax
import jax.numpy as jnp
from jax import lax
from jax.experimental import pallas as pl
from jax.experimental.pallas import tpu as pltpu
from jax.experimental.pallas import tpu_sc as plsc

_BM = 512


def _route_meta(token_per_expert, n_rows, n_experts):
    nb = n_rows // _BM
    t_max = nb + n_experts - 1
    off = jnp.concatenate([
        jnp.zeros((1,), jnp.int32),
        jnp.cumsum(token_per_expert).astype(jnp.int32),
    ])
    b = jnp.arange(nb, dtype=jnp.int32)[:, None]
    seg_lo = jnp.maximum(off[:-1][None, :], b * _BM)
    seg_hi = jnp.minimum(off[1:][None, :], (b + 1) * _BM)
    valid = seg_lo < seg_hi
    vflat = valid.reshape(-1)
    slot = jnp.where(vflat, jnp.cumsum(vflat) - 1, t_max).astype(jnp.int32)

    def scat(vals, fill):
        buf = jnp.full((t_max + 1,), fill, jnp.int32)
        return buf.at[slot].set(vals.reshape(-1).astype(jnp.int32))[:t_max]

    blk = scat(jnp.broadcast_to(b, valid.shape), nb - 1)
    expt = scat(jnp.broadcast_to(jnp.arange(n_experts, dtype=jnp.int32)[None, :],
                                 valid.shape), n_experts - 1)
    lo = scat(seg_lo - b * _BM, 0)
    hi = scat(seg_hi - b * _BM, 0)
    first = scat(valid & (jnp.cumsum(valid, axis=1) == 1), 0)
    return blk, expt, lo, hi, first


def _tc_grouped_mlp(xg, probs_sorted, w_gate, w_up, w_down,
                    blk, expt, lo, hi, first):
    n_rows, h = xg.shape
    n_experts, ff, _ = w_gate.shape
    nb = n_rows // _BM
    t_max = nb + n_experts - 1
    probs3 = probs_sorted.reshape(nb, 1, _BM)

    def body(blk_r, expt_r, lo_r, hi_r, first_r, x_r, p_r, wg_r, wu_r, wd_r,
             o_r):
        t = pl.program_id(0)
        rows = lax.broadcasted_iota(jnp.int32, (_BM, 1), 0)
        mask = (rows >= lo_r[t]) & (rows < hi_r[t])
        x = x_r[...]
        g = lax.dot_general(x, wg_r[0], (((1,), (1,)), ((), ())),
                            preferred_element_type=jnp.float32)
        u = lax.dot_general(x, wu_r[0], (((1,), (1,)), ((), ())),
                            preferred_element_type=jnp.float32)
        act = u * (g * jax.nn.sigmoid(g))
        d = lax.dot_general(act, wd_r[0], (((1,), (1,)), ((), ())),
                            preferred_element_type=jnp.float32)
        d = jnp.where(mask, d * p_r[0, 0, :].reshape(_BM, 1), 0.0)
        is_init = first_r[t] == 1

        @pl.when(is_init)
        def _():
            o_r[...] = d

        @pl.when(jnp.logical_not(is_init))
        def _():
            o_r[...] += d

    grid_spec = pltpu.PrefetchScalarGridSpec(
        num_scalar_prefetch=5,
        grid=(t_max,),
        in_specs=[
            pl.BlockSpec((_BM, h), lambda t, blk, expt, lo, hi, first: (blk[t], 0)),
            pl.BlockSpec((1, 1, _BM), lambda t, blk, expt, lo, hi, first: (blk[t], 0, 0)),
            pl.BlockSpec((1, ff, h), lambda t, blk, expt, lo, hi, first: (expt[t], 0, 0)),
            pl.BlockSpec((1, ff, h), lambda t, blk, expt, lo, hi, first: (expt[t], 0, 0)),
            pl.BlockSpec((1, h, ff), lambda t, blk, expt, lo, hi, first: (expt[t], 0, 0)),
        ],
        out_specs=pl.BlockSpec((_BM, h), lambda t, blk, expt, lo, hi, first: (blk[t], 0)),
    )
    return pl.pallas_call(
        body,
        grid_spec=grid_spec,
        out_shape=jax.ShapeDtypeStruct((n_rows, h), jnp.float32),
        compiler_params=pltpu.CompilerParams(
            dimension_semantics=("arbitrary",)),
    )(blk, expt, lo, hi, first, xg, probs3, w_gate, w_up, w_down)


def _sc_gather_rows(table, idx):
    n_rows = idx.shape[0]
    h = table.shape[1]
    info = plsc.get_sparse_core_info()
    nw = info.num_cores * info.num_subcores
    bpw = n_rows // nw
    ch = min(64, bpw)
    nch = bpw // ch
    idx3 = idx.reshape(nw, nch, ch)
    mesh = plsc.VectorSubcoreMesh(core_axis_name="c", subcore_axis_name="s")

    @functools.partial(
        pl.kernel,
        mesh=mesh,
        out_type=jax.ShapeDtypeStruct((n_rows, h), jnp.float32),
        scratch_types=[
            pltpu.VMEM((nch, ch), jnp.int32),
            pltpu.VMEM((ch, h), jnp.float32),
            pltpu.SemaphoreType.DMA,
        ],
    )
    def k(table_hbm, idx_hbm, out_hbm, idx_v, rows_v, sem):
        wid = lax.axis_index("s") * info.num_cores + lax.axis_index("c")
        base = wid * bpw
        pltpu.sync_copy(idx_hbm.at[wid], idx_v)
        for c in range(nch):
            pltpu.async_copy(table_hbm.at[idx_v.at[c]], rows_v, sem).wait()
            pltpu.sync_copy(rows_v, out_hbm.at[pl.ds(base + c * ch, ch)])

    return k(table, idx3)


def _sc_combine(down, pos_a, pos_b):
    s = pos_a.shape[0]
    h = down.shape[1]
    info = plsc.get_sparse_core_info()
    nw = info.num_cores * info.num_subcores
    bpw = s // nw
    ch = min(16, bpw)
    nch = bpw // ch
    nvec = h // 16
    pa3 = pos_a.reshape(nw, nch, ch)
    pb3 = pos_b.reshape(nw, nch, ch)
    mesh = plsc.VectorSubcoreMesh(core_axis_name="c", subcore_axis_name="s")

    @functools.partial(
        pl.kernel,
        mesh=mesh,
        out_type=jax.ShapeDtypeStruct((s, h), jnp.float32),
        scratch_types=[
            pltpu.VMEM((nch, ch), jnp.int32),
            pltpu.VMEM((nch, ch), jnp.int32),
            pltpu.VMEM((ch, h), jnp.float32),
            pltpu.VMEM((ch, h), jnp.float32),
            pltpu.VMEM((ch, h), jnp.float32),
            pltpu.VMEM((ch, h), jnp.float32),
            pltpu.SemaphoreType.DMA,
            pltpu.SemaphoreType.DMA,
            pltpu.SemaphoreType.DMA,
            pltpu.SemaphoreType.DMA,
        ],
    )
    def k(down_hbm, pa_hbm, pb_hbm, out_hbm, ia, ib,
          ra0, rb0, ra1, rb1, sa0, sb0, sa1, sb1):
        wid = lax.axis_index("s") * info.num_cores + lax.axis_index("c")
        base = wid * bpw
        pltpu.sync_copy(pa_hbm.at[wid], ia)
        pltpu.sync_copy(pb_hbm.at[wid], ib)
        ras = (ra0, ra1)
        rbs = (rb0, rb1)
        sas = (sa0, sa1)
        sbs = (sb0, sb1)
        cpa = [None] * nch
        cpb = [None] * nch
        cpa[0] = pltpu.async_copy(down_hbm.at[ia.at[0]], ras[0], sas[0])
        cpb[0] = pltpu.async_copy(down_hbm.at[ib.at[0]], rbs[0], sbs[0])
        for c in range(nch):
            cpa[c].wait()
            cpb[c].wait()
            if c + 1 < nch:
                j = (c + 1) % 2
                cpa[c + 1] = pltpu.async_copy(down_hbm.at[ia.at[c + 1]], ras[j], sas[j])
                cpb[c + 1] = pltpu.async_copy(down_hbm.at[ib.at[c + 1]], rbs[j], sbs[j])
            ra = ras[c % 2]
            rb = rbs[c % 2]

            def add_row(r, carry):
                for j in range(nvec):
                    ra[r, pl.ds(j * 16, 16)] = (
                        ra[r, pl.ds(j * 16, 16)] + rb[r, pl.ds(j * 16, 16)])
                return carry

            lax.fori_loop(0, ch, add_row, 0)
            pltpu.sync_copy(ra, out_hbm.at[pl.ds(base + c * ch, ch)])

    return k(down, pa3, pb3)


def kernel(hidden_states, router_weights, selected_experts, token_per_expert,
           W_gate, W_up, W_down):
    s, h = hidden_states.shape
    k_ = router_weights.shape[1]
    n_experts = W_gate.shape[0]
    n_rows = k_ * s
    nb = n_rows // _BM

    dup = selected_experts.T.reshape(-1)
    sort_idx = jnp.argsort(dup, stable=True).astype(jnp.int32)
    pos = jnp.argsort(sort_idx).astype(jnp.int32)
    src_token = (sort_idx % s).astype(jnp.int32)
    probs_sorted = jnp.take(router_weights.T.reshape(-1).astype(jnp.float32),
                            sort_idx)
    blk, expt, lo, hi, first = _route_meta(token_per_expert, n_rows, n_experts)

    grouped = _sc_gather_rows(hidden_states, src_token)
    down = _tc_grouped_mlp(grouped, probs_sorted, W_gate, W_up, W_down,
                           blk, expt, lo, hi, first)
    return _sc_combine(down, pos[:s], pos[s:])
